# baseline (speedup 1.0000x reference)
import functools

import jax
import jax.numpy as jnp
from jax import lax
from jax.experimental import pallas as pl
from jax.experimental.pallas import tpu as pltpu
from jax.experimental.pallas import tpu_sc as plsc

E = 8
K = 2
T = 2
D = 768
F = 768
N = 2048

BN = 2048

_NEG_INF = float("-inf")


def _logits_body(x_ref, wg_ref, out_ref):
    lg = jnp.dot(x_ref[...], wg_ref[...], preferred_element_type=jnp.float32)
    out_ref[...] = lg.T


def _logits(x, Wg2):
    return pl.pallas_call(
        _logits_body,
        out_shape=jax.ShapeDtypeStruct((T * E, N), jnp.float32),
    )(x, Wg2)


_NW = 16
_CHUNK = N // _NW
_L = 16


def _route_body(lt_hbm, gates_hbm, lt_v, gv):
    wid = lax.axis_index("s") * 2 + lax.axis_index("c")
    base = wid * _CHUNK
    pltpu.sync_copy(lt_hbm.at[:, pl.ds(base, _CHUNK)], lt_v)
    for j in range(_CHUNK // _L):
        for t in range(T):
            v = [lt_v[t * E + e, pl.ds(j * _L, _L)] for e in range(E)]
            m1 = v[0]
            for e in range(1, E):
                m1 = jnp.maximum(m1, v[e])
            i1 = jnp.full((_L,), E, jnp.int32)
            for e in range(E - 1, -1, -1):
                i1 = jnp.where(v[e] == m1, e, i1)
            m2 = jnp.full((_L,), _NEG_INF, jnp.float32)
            for e in range(E):
                m2 = jnp.where(i1 == e, m2, jnp.maximum(m2, v[e]))
            i2 = jnp.full((_L,), E, jnp.int32)
            for e in range(E - 1, -1, -1):
                i2 = jnp.where((v[e] == m2) & (i1 != e), e, i2)
            d = jnp.exp(m2 - m1)
            g1 = 1.0 / (1.0 + d)
            g2 = d * g1
            for e in range(E):
                ge = jnp.where(i1 == e, g1, jnp.where(i2 == e, g2, 0.0))
                gv[t * E + e, pl.ds(j * _L, _L)] = ge
    pltpu.sync_copy(gv, gates_hbm.at[:, pl.ds(base, _CHUNK)])


def _route(logitsT):
    mesh = plsc.VectorSubcoreMesh(core_axis_name="c", subcore_axis_name="s")
    body = functools.partial(
        pl.kernel,
        out_type=jax.ShapeDtypeStruct((T * E, N), jnp.float32),
        mesh=mesh,
        scratch_types=[
            pltpu.VMEM((T * E, _CHUNK), jnp.float32),
            pltpu.VMEM((T * E, _CHUNK), jnp.float32),
        ],
    )

    @body
    def routed(lt_hbm, gates_hbm, lt_v, gv):
        wid = lax.axis_index("s") * 2 + lax.axis_index("c")

        @pl.when(wid < _NW)
        def _():
            _route_body(lt_hbm, gates_hbm, lt_v, gv)

    return routed(logitsT)


P = 2


def _ffn_body(x_ref, g_ref, w1_ref, w2_ref, out_ref, h2_ref):
    p = pl.program_id(0)
    xb = x_ref[...].astype(jnp.bfloat16)

    row = lax.broadcasted_iota(jnp.int32, (T * E, T * P), 0)
    cid = lax.broadcasted_iota(jnp.int32, (T * E, T * P), 1)
    sel = (row == (cid // P) * E + p * P + (cid % P))
    gcols = lax.dot_general(
        g_ref[...], sel.astype(jnp.float32),
        dimension_numbers=(((0,), (0,)), ((), ())),
        preferred_element_type=jnp.float32)

    for q in range(P):
        h = jnp.dot(xb, w1_ref[q].astype(jnp.bfloat16),
                    preferred_element_type=jnp.float32)
        h = jnp.maximum(h, 0.01 * h)
        for t in range(T):
            h2_ref[t, :, q * F:(q + 1) * F] = (
                gcols[:, t * P + q:t * P + q + 1] * h).astype(jnp.bfloat16)

    w2b = jnp.reshape(w2_ref[...].astype(jnp.bfloat16), (P * F, D))
    for t in range(T):
        y = jnp.dot(h2_ref[t], w2b, preferred_element_type=jnp.float32)

        @pl.when(p == 0)
        def _first():
            out_ref[t] = y

        @pl.when(p != 0)
        def _rest():
            out_ref[t] += y


def _ffn(x, gatesT, W1, W2):
    grid = (E // P,)
    return pl.pallas_call(
        _ffn_body,
        grid=grid,
        in_specs=[
            pl.BlockSpec((BN, D), lambda p: (0, 0)),
            pl.BlockSpec((T * E, BN), lambda p: (0, 0)),
            pl.BlockSpec((P, D, F), lambda p: (p, 0, 0)),
            pl.BlockSpec((P, F, D), lambda p: (p, 0, 0)),
        ],
        out_specs=pl.BlockSpec((T, BN, D), lambda p: (0, 0, 0)),
        out_shape=jax.ShapeDtypeStruct((T, N, D), jnp.float32),
        scratch_shapes=[
            pltpu.VMEM((T, BN, P * F), jnp.bfloat16),
        ],
    )(x, gatesT, W1, W2)


def kernel(x, Wg, W1, W2):
    Wg2 = Wg.transpose(1, 0, 2).reshape(D, T * E)
    logitsT = _logits(x, Wg2)
    gatesT = _route(logitsT)
    return _ffn(x, gatesT, W1, W2)


# speedup vs baseline: 1.3539x; 1.3539x over previous
import functools

import jax
import jax.numpy as jnp
from jax import lax
from jax.experimental import pallas as pl
from jax.experimental.pallas import tpu as pltpu
from jax.experimental.pallas import tpu_sc as plsc

E = 8
K = 2
T = 2
D = 768
F = 768
N = 2048

BN = 2048

_NEG_INF = float("-inf")


def _logits_body(x_ref, wg_ref, out_ref):
    lg = jnp.dot(x_ref[...], wg_ref[...], preferred_element_type=jnp.float32)
    out_ref[...] = lg.T


def _logits(x, Wg2):
    return pl.pallas_call(
        _logits_body,
        out_shape=jax.ShapeDtypeStruct((T * E, N), jnp.float32),
    )(x, Wg2)


_NW = 16
_CHUNK = N // _NW
_L = 16


def _route_body(lt_hbm, gates_hbm, lt_v, gv):
    wid = lax.axis_index("s") * 2 + lax.axis_index("c")
    base = wid * _CHUNK
    pltpu.sync_copy(lt_hbm.at[:, pl.ds(base, _CHUNK)], lt_v)
    for j in range(_CHUNK // _L):
        for t in range(T):
            v = [lt_v[t * E + e, pl.ds(j * _L, _L)] for e in range(E)]
            m1 = v[0]
            for e in range(1, E):
                m1 = jnp.maximum(m1, v[e])
            i1 = jnp.full((_L,), E, jnp.int32)
            for e in range(E - 1, -1, -1):
                i1 = jnp.where(v[e] == m1, e, i1)
            m2 = jnp.full((_L,), _NEG_INF, jnp.float32)
            for e in range(E):
                m2 = jnp.where(i1 == e, m2, jnp.maximum(m2, v[e]))
            i2 = jnp.full((_L,), E, jnp.int32)
            for e in range(E - 1, -1, -1):
                i2 = jnp.where((v[e] == m2) & (i1 != e), e, i2)
            d = jnp.exp(m2 - m1)
            g1 = 1.0 / (1.0 + d)
            g2 = d * g1
            for e in range(E):
                ge = jnp.where(i1 == e, g1, jnp.where(i2 == e, g2, 0.0))
                gv[t * E + e, pl.ds(j * _L, _L)] = ge
    pltpu.sync_copy(gv, gates_hbm.at[:, pl.ds(base, _CHUNK)])


def _route(logitsT):
    mesh = plsc.VectorSubcoreMesh(core_axis_name="c", subcore_axis_name="s")
    body = functools.partial(
        pl.kernel,
        out_type=jax.ShapeDtypeStruct((T * E, N), jnp.float32),
        mesh=mesh,
        scratch_types=[
            pltpu.VMEM((T * E, _CHUNK), jnp.float32),
            pltpu.VMEM((T * E, _CHUNK), jnp.float32),
        ],
    )

    @body
    def routed(lt_hbm, gates_hbm, lt_v, gv):
        wid = lax.axis_index("s") * 2 + lax.axis_index("c")

        @pl.when(wid < _NW)
        def _():
            _route_body(lt_hbm, gates_hbm, lt_v, gv)

    return routed(logitsT)


def _ffn_body(x_ref, g_ref, w1_ref, w2_ref, out_ref):
    e = pl.program_id(1)
    xb = x_ref[...].astype(jnp.bfloat16)
    h = jnp.dot(xb, w1_ref[0].astype(jnp.bfloat16),
                preferred_element_type=jnp.float32)
    h = jnp.maximum(h, 0.01 * h)
    y = jnp.dot(h.astype(jnp.bfloat16), w2_ref[0].astype(jnp.bfloat16),
                preferred_element_type=jnp.float32)

    col = lax.broadcasted_iota(jnp.int32, (T * E, T), 0)
    sel = (col == (lax.broadcasted_iota(jnp.int32, (T * E, T), 1) * E + e))
    gcols = lax.dot_general(
        g_ref[...], sel.astype(jnp.float32),
        dimension_numbers=(((0,), (0,)), ((), ())),
        preferred_element_type=jnp.float32)

    @pl.when(e == 0)
    def _first():
        for t in range(T):
            out_ref[t] = gcols[:, t:t + 1] * y

    @pl.when(e != 0)
    def _rest():
        for t in range(T):
            out_ref[t] += gcols[:, t:t + 1] * y


def _ffn(x, gatesT, W1, W2):
    grid = (N // BN, E)
    return pl.pallas_call(
        _ffn_body,
        grid=grid,
        in_specs=[
            pl.BlockSpec((BN, D), lambda i, e: (i, 0)),
            pl.BlockSpec((T * E, BN), lambda i, e: (0, i)),
            pl.BlockSpec((1, D, F), lambda i, e: (e, 0, 0)),
            pl.BlockSpec((1, F, D), lambda i, e: (e, 0, 0)),
        ],
        out_specs=pl.BlockSpec((T, BN, D), lambda i, e: (0, i, 0)),
        out_shape=jax.ShapeDtypeStruct((T, N, D), jnp.float32),
    )(x, gatesT, W1, W2)


def kernel(x, Wg, W1, W2):
    Wg2 = Wg.transpose(1, 0, 2).reshape(D, T * E)
    logitsT = _logits(x, Wg2)
    gatesT = _route(logitsT)
    return _ffn(x, gatesT, W1, W2)
